# Initial kernel scaffold; baseline (speedup 1.0000x reference)
import jax
import jax.numpy as jnp
from jax.experimental import pallas as pl


def kernel(x, edge_index, batch, W1, b1, W2, b2, W3, b3, Wl1, bl1, Wl2, bl2):
    raise NotImplementedError("write your pallas kernel here")


# speedup vs baseline: 6.8523x; 6.8523x over previous
import functools

import jax
import jax.numpy as jnp
from jax import lax
from jax.experimental import pallas as pl
from jax.experimental.pallas import tpu as pltpu
from jax.experimental.pallas import tpu_sc as plsc

_N = 10000
_E = 320000
_D = 128
_G = 16

_NP = 10240
_NC = 2
_NS = 16
_NW = _NC * _NS
_C = 128
_CH = 80
_EP = _NW * _CH * _C
_RPS = _NP // _NS

_BLK = 1024
_NB = _NP // _BLK


def _mesh():
    return plsc.VectorSubcoreMesh(core_axis_name="c", subcore_axis_name="s",
                                  num_cores=_NC, num_subcores=_NS)


@functools.cache
def _deg_kernel():
    @functools.partial(
        pl.kernel,
        out_type=jax.ShapeDtypeStruct((_NC, _NP, _D), jnp.float32),
        mesh=_mesh(),
        scratch_types=[
            pltpu.VMEM((_CH, _C), jnp.int32),
            pltpu.VMEM((_C, _D), jnp.float32),
            pltpu.VMEM_SHARED((_NP, _D), jnp.float32),
        ],
    )
    def deg_k(dst_hbm, ones_hbm, zeros_hbm, out_hbm, dst_v, ones_v, acc_sh):
        cid = lax.axis_index("c")
        sid = lax.axis_index("s")
        wid = cid * _NS + sid
        r0 = sid * _RPS
        pltpu.sync_copy(zeros_hbm.at[pl.ds(r0, _RPS)], acc_sh.at[pl.ds(r0, _RPS)])
        pltpu.sync_copy(dst_hbm.at[wid], dst_v)
        pltpu.sync_copy(ones_hbm, ones_v)
        plsc.subcore_barrier()

        def body(ch, carry):
            pltpu.sync_copy(ones_v, acc_sh.at[dst_v.at[ch]], add=True)
            return carry

        lax.fori_loop(0, _CH, body, 0)
        plsc.subcore_barrier()
        pltpu.sync_copy(acc_sh.at[pl.ds(r0, _RPS)], out_hbm.at[cid, pl.ds(r0, _RPS)])

    return deg_k


@functools.cache
def _agg_kernel():
    @functools.partial(
        pl.kernel,
        out_type=jax.ShapeDtypeStruct((_NC, _NP, _D), jnp.float32),
        mesh=_mesh(),
        scratch_types=[
            pltpu.VMEM((_CH, _C), jnp.int32),
            pltpu.VMEM((_CH, _C), jnp.int32),
            pltpu.VMEM((_C, _D), jnp.float32),
            pltpu.VMEM_SHARED((_NP, _D), jnp.float32),
            pltpu.SemaphoreType.DMA,
        ],
    )
    def agg_k(hs_hbm, src_hbm, dst_hbm, zeros_hbm, out_hbm,
              src_v, dst_v, rows_v, acc_sh, sem):
        cid = lax.axis_index("c")
        sid = lax.axis_index("s")
        wid = cid * _NS + sid
        r0 = sid * _RPS
        pltpu.sync_copy(zeros_hbm.at[pl.ds(r0, _RPS)], acc_sh.at[pl.ds(r0, _RPS)])
        pltpu.sync_copy(src_hbm.at[wid], src_v)
        pltpu.sync_copy(dst_hbm.at[wid], dst_v)
        plsc.subcore_barrier()

        def body(ch, carry):
            pltpu.async_copy(hs_hbm.at[src_v.at[ch]], rows_v, sem).wait()
            pltpu.sync_copy(rows_v, acc_sh.at[dst_v.at[ch]], add=True)
            return carry

        lax.fori_loop(0, _CH, body, 0)
        plsc.subcore_barrier()
        pltpu.sync_copy(acc_sh.at[pl.ds(r0, _RPS)], out_hbm.at[cid, pl.ds(r0, _RPS)])

    return agg_k


@functools.cache
def _tc1():
    def body(p0, p1, x, w, dinv_ref, hs_ref):
        deg = p0[:, 0:1] + p1[:, 0:1] + 1.0
        dinv = 1.0 / jnp.sqrt(deg)
        dinv_ref[...] = dinv
        hs_ref[...] = dinv * jnp.dot(x[...], w[...],
                                     preferred_element_type=jnp.float32)

    return pl.pallas_call(
        body,
        grid=(_NB,),
        in_specs=[
            pl.BlockSpec((_BLK, _D), lambda i: (i, 0)),
            pl.BlockSpec((_BLK, _D), lambda i: (i, 0)),
            pl.BlockSpec((_BLK, _D), lambda i: (i, 0)),
            pl.BlockSpec((_D, _D), lambda i: (0, 0)),
        ],
        out_specs=[
            pl.BlockSpec((_BLK, 1), lambda i: (i, 0)),
            pl.BlockSpec((_BLK, _D), lambda i: (i, 0)),
        ],
        out_shape=[
            jax.ShapeDtypeStruct((_NP, 1), jnp.float32),
            jax.ShapeDtypeStruct((_NP, _D), jnp.float32),
        ],
    )


@functools.cache
def _tc2():
    def body(p0, p1, hs, dinv, b, w, out_ref):
        i = pl.program_id(0)
        rid = lax.broadcasted_iota(jnp.int32, (_BLK, 1), 0) + i * _BLK
        dv = dinv[...]
        h = dv * (p0[...] + p1[...] + hs[...]) + b[...]
        h = jnp.where(rid < _N, jnp.maximum(h, 0.0), 0.0)
        out_ref[...] = dv * jnp.dot(h, w[...], preferred_element_type=jnp.float32)

    return pl.pallas_call(
        body,
        grid=(_NB,),
        in_specs=[
            pl.BlockSpec((_BLK, _D), lambda i: (i, 0)),
            pl.BlockSpec((_BLK, _D), lambda i: (i, 0)),
            pl.BlockSpec((_BLK, _D), lambda i: (i, 0)),
            pl.BlockSpec((_BLK, 1), lambda i: (i, 0)),
            pl.BlockSpec((1, _D), lambda i: (0, 0)),
            pl.BlockSpec((_D, _D), lambda i: (0, 0)),
        ],
        out_specs=pl.BlockSpec((_BLK, _D), lambda i: (i, 0)),
        out_shape=jax.ShapeDtypeStruct((_NP, _D), jnp.float32),
    )


@functools.cache
def _tc3():
    def body(p0, p1, hs, dinv, b, bat, wl1, bl1, wl2, bl2, out_ref, g_ref):
        i = pl.program_id(0)

        @pl.when(i == 0)
        def _init():
            g_ref[...] = jnp.full((_G, _D), -jnp.inf, jnp.float32)

        rid = lax.broadcasted_iota(jnp.int32, (_BLK, 1), 0) + i * _BLK
        h = dinv[...] * (p0[...] + p1[...] + hs[...]) + b[...]
        h = jnp.where(rid < _N, jnp.maximum(h, 0.0), -jnp.inf)
        bv = bat[...]
        parts = [jnp.max(jnp.where(bv == g, h, -jnp.inf), axis=0, keepdims=True)
                 for g in range(_G)]
        g_ref[...] = jnp.maximum(g_ref[...], jnp.concatenate(parts, axis=0))

        @pl.when(i == _NB - 1)
        def _finish():
            gg = g_ref[...]
            z = jnp.maximum(
                jnp.dot(gg, wl1[...], preferred_element_type=jnp.float32)
                + bl1[...], 0.0)
            o = jnp.dot(z, wl2[...], preferred_element_type=jnp.float32) + bl2[...]
            out_ref[...] = 1.0 / (1.0 + jnp.exp(-o))

    return pl.pallas_call(
        body,
        grid=(_NB,),
        in_specs=[
            pl.BlockSpec((_BLK, _D), lambda i: (i, 0)),
            pl.BlockSpec((_BLK, _D), lambda i: (i, 0)),
            pl.BlockSpec((_BLK, _D), lambda i: (i, 0)),
            pl.BlockSpec((_BLK, 1), lambda i: (i, 0)),
            pl.BlockSpec((1, _D), lambda i: (0, 0)),
            pl.BlockSpec((_BLK, 1), lambda i: (i, 0)),
            pl.BlockSpec((_D, 256), lambda i: (0, 0)),
            pl.BlockSpec((1, 256), lambda i: (0, 0)),
            pl.BlockSpec((256, 10), lambda i: (0, 0)),
            pl.BlockSpec((1, 10), lambda i: (0, 0)),
        ],
        out_specs=pl.BlockSpec((_G, 10), lambda i: (0, 0)),
        out_shape=jax.ShapeDtypeStruct((_G, 10), jnp.float32),
        scratch_shapes=[pltpu.VMEM((_G, _D), jnp.float32)],
    )


def kernel(x, edge_index, batch, W1, b1, W2, b2, W3, b3, Wl1, bl1, Wl2, bl2):
    f32 = jnp.float32
    x_p = jnp.pad(x, ((0, _NP - _N), (0, 0)))
    ei = edge_index.astype(jnp.int32)
    pad_e = jnp.full((_EP - _E,), _N, jnp.int32)
    src = jnp.concatenate([ei[0], pad_e]).reshape(_NW, _CH, _C)
    dst = jnp.concatenate([ei[1], pad_e]).reshape(_NW, _CH, _C)
    bat_p = jnp.concatenate(
        [batch.astype(jnp.int32), jnp.full((_NP - _N,), _G, jnp.int32)]
    ).reshape(_NP, 1)
    ones = jnp.ones((_C, _D), f32)
    zeros = jnp.zeros((_NP, _D), f32)

    W1p = jnp.pad(W1, ((0, 0), (0, _D - W1.shape[1])))
    W2p = jnp.pad(W2, ((0, _D - W2.shape[0]), (0, _D - W2.shape[1])))
    W3p = jnp.pad(W3, ((0, _D - W3.shape[0]), (0, _D - W3.shape[1])))
    Wl1p = jnp.pad(Wl1, ((0, _D - Wl1.shape[0]), (0, 0)))
    b1p = jnp.pad(b1, (0, _D - b1.shape[0])).reshape(1, _D)
    b2p = jnp.pad(b2, (0, _D - b2.shape[0])).reshape(1, _D)
    b3p = jnp.pad(b3, (0, _D - b3.shape[0])).reshape(1, _D)

    degp = _deg_kernel()(dst, ones, zeros)
    dinv, hs1 = _tc1()(degp[0], degp[1], x_p, W1p)
    p1 = _agg_kernel()(hs1, src, dst, zeros)
    hs2 = _tc2()(p1[0], p1[1], hs1, dinv, b1p, W2p)
    p2 = _agg_kernel()(hs2, src, dst, zeros)
    hs3 = _tc2()(p2[0], p2[1], hs2, dinv, b2p, W3p)
    p3 = _agg_kernel()(hs3, src, dst, zeros)
    out = _tc3()(p3[0], p3[1], hs3, dinv, b3p, bat_p,
                 Wl1p, bl1.reshape(1, -1), Wl2, bl2.reshape(1, -1))
    return out


# speedup vs baseline: 7.3782x; 1.0767x over previous
import functools

import jax
import jax.numpy as jnp
from jax import lax
from jax.experimental import pallas as pl
from jax.experimental.pallas import tpu as pltpu
from jax.experimental.pallas import tpu_sc as plsc

_N = 10000
_E = 320000
_D = 128
_G = 16

_NP = 10240
_NC = 2
_NS = 16
_NW = _NC * _NS
_C = 128
_CH = 80
_EP = _NW * _CH * _C
_RPS = _NP // _NS
_R = 2
_HC = 40

_BLK = 1024
_NB = _NP // _BLK


def _mesh():
    return plsc.VectorSubcoreMesh(core_axis_name="c", subcore_axis_name="s",
                                  num_cores=_NC, num_subcores=_NS)


@functools.cache
def _deg_kernel():
    @functools.partial(
        pl.kernel,
        out_type=jax.ShapeDtypeStruct((_NC, _NP, _D), jnp.float32),
        mesh=_mesh(),
        scratch_types=[
            pltpu.VMEM((_CH, _C), jnp.int32),
            pltpu.VMEM((_C, _D), jnp.float32),
            pltpu.VMEM_SHARED((_NP, _D), jnp.float32),
            pltpu.SemaphoreType.DMA,
        ],
    )
    def deg_k(dst_hbm, ones_hbm, zeros_hbm, out_hbm, dst_v, ones_v, acc_sh, sem):
        cid = lax.axis_index("c")
        sid = lax.axis_index("s")
        wid = cid * _NS + sid
        r0 = sid * _RPS
        pltpu.sync_copy(zeros_hbm.at[pl.ds(r0, _RPS)], acc_sh.at[pl.ds(r0, _RPS)])
        pltpu.sync_copy(dst_hbm.at[wid], dst_v)
        pltpu.sync_copy(ones_hbm, ones_v)
        plsc.subcore_barrier()

        def body(ch, carry):
            pltpu.sync_copy(ones_v, acc_sh.at[dst_v.at[ch]], add=True)
            return carry

        lax.fori_loop(0, _CH, body, 0)
        plsc.subcore_barrier()
        pltpu.sync_copy(acc_sh.at[pl.ds(r0, _RPS)], out_hbm.at[cid, pl.ds(r0, _RPS)])

    return deg_k


@functools.cache
def _agg_kernel(d):
    del d

    @functools.partial(
        pl.kernel,
        out_type=jax.ShapeDtypeStruct((_NC, _NP, _D), jnp.float32),
        mesh=_mesh(),
        scratch_types=[
            pltpu.VMEM((_HC, _C), jnp.int32),
            pltpu.VMEM((_HC, _C), jnp.int32),
            pltpu.VMEM((_R, _C, _D), jnp.float32),
            pltpu.VMEM_SHARED((_NP, _D), jnp.float32),
            [pltpu.SemaphoreType.DMA] * _R,
            [pltpu.SemaphoreType.DMA] * _R,
        ],
    )
    def agg_k(hs_hbm, src_hbm, dst_hbm, zeros_hbm, out_hbm,
              src_v, dst_v, rows_v, acc_sh, gsem, ssem):
        cid = lax.axis_index("c")
        sid = lax.axis_index("s")
        wid = cid * _NS + sid
        r0 = sid * _RPS
        pltpu.sync_copy(zeros_hbm.at[pl.ds(r0, _RPS)], acc_sh.at[pl.ds(r0, _RPS)])
        plsc.subcore_barrier()

        def gather(ch, b):
            return pltpu.async_copy(hs_hbm.at[src_v.at[ch]], rows_v.at[b], gsem[b])

        def scatter(ch, b):
            return pltpu.async_copy(rows_v.at[b], acc_sh.at[dst_v.at[ch]],
                                    ssem[b], add=True)

        for half in range(_CH // _HC):
            pltpu.sync_copy(src_hbm.at[wid, pl.ds(half * _HC, _HC)], src_v)
            pltpu.sync_copy(dst_hbm.at[wid, pl.ds(half * _HC, _HC)], dst_v)
            for b in range(_R):
                gather(b, b)

            def body(i, carry):
                for b in range(_R):
                    ch = i * _R + b
                    pltpu.make_async_copy(hs_hbm.at[src_v.at[ch]],
                                          rows_v.at[b], gsem[b]).wait()
                    scatter(ch, b)
                    pb = b - 1 if b else _R - 1
                    pch = ch - 1

                    @pl.when(pch >= 0)
                    def _():
                        pltpu.make_async_copy(
                            rows_v.at[pb],
                            acc_sh.at[dst_v.at[lax.max(pch, 0)]],
                            ssem[pb]).wait()

                        @pl.when(pch + _R < _HC)
                        def _():
                            gather(pch + _R, pb)
                return carry

            lax.fori_loop(0, _HC // _R, body, 0)
            pltpu.make_async_copy(rows_v.at[_R - 1],
                                  acc_sh.at[dst_v.at[_HC - 1]],
                                  ssem[_R - 1]).wait()
        plsc.subcore_barrier()
        pltpu.sync_copy(acc_sh.at[pl.ds(r0, _RPS)], out_hbm.at[cid, pl.ds(r0, _RPS)])

    return agg_k


@functools.cache
def _tc1():
    def body(p0, p1, x, w, dinv_ref, hs_ref):
        deg = p0[:, 0:1] + p1[:, 0:1] + 1.0
        dinv = 1.0 / jnp.sqrt(deg)
        dinv_ref[...] = dinv
        hs_ref[...] = dinv * jnp.dot(x[...], w[...],
                                     preferred_element_type=jnp.float32)

    return pl.pallas_call(
        body,
        grid=(_NB,),
        in_specs=[
            pl.BlockSpec((_BLK, _D), lambda i: (i, 0)),
            pl.BlockSpec((_BLK, _D), lambda i: (i, 0)),
            pl.BlockSpec((_BLK, _D), lambda i: (i, 0)),
            pl.BlockSpec((_D, _D), lambda i: (0, 0)),
        ],
        out_specs=[
            pl.BlockSpec((_BLK, 1), lambda i: (i, 0)),
            pl.BlockSpec((_BLK, _D), lambda i: (i, 0)),
        ],
        out_shape=[
            jax.ShapeDtypeStruct((_NP, 1), jnp.float32),
            jax.ShapeDtypeStruct((_NP, _D), jnp.float32),
        ],
    )


@functools.cache
def _tc2():
    def body(p0, p1, hs, dinv, b, w, out_ref):
        i = pl.program_id(0)
        rid = lax.broadcasted_iota(jnp.int32, (_BLK, 1), 0) + i * _BLK
        dv = dinv[...]
        h = dv * (p0[...] + p1[...] + hs[...]) + b[...]
        h = jnp.where(rid < _N, jnp.maximum(h, 0.0), 0.0)
        out_ref[...] = dv * jnp.dot(h, w[...], preferred_element_type=jnp.float32)

    return pl.pallas_call(
        body,
        grid=(_NB,),
        in_specs=[
            pl.BlockSpec((_BLK, _D), lambda i: (i, 0)),
            pl.BlockSpec((_BLK, _D), lambda i: (i, 0)),
            pl.BlockSpec((_BLK, _D), lambda i: (i, 0)),
            pl.BlockSpec((_BLK, 1), lambda i: (i, 0)),
            pl.BlockSpec((1, _D), lambda i: (0, 0)),
            pl.BlockSpec((_D, _D), lambda i: (0, 0)),
        ],
        out_specs=pl.BlockSpec((_BLK, _D), lambda i: (i, 0)),
        out_shape=jax.ShapeDtypeStruct((_NP, _D), jnp.float32),
    )


@functools.cache
def _tc3():
    def body(p0, p1, hs, dinv, b, bat, wl1, bl1, wl2, bl2, out_ref, g_ref):
        i = pl.program_id(0)

        @pl.when(i == 0)
        def _init():
            g_ref[...] = jnp.full((_G, _D), -jnp.inf, jnp.float32)

        rid = lax.broadcasted_iota(jnp.int32, (_BLK, 1), 0) + i * _BLK
        h = dinv[...] * (p0[...] + p1[...] + hs[...]) + b[...]
        h = jnp.where(rid < _N, jnp.maximum(h, 0.0), -jnp.inf)
        bv = bat[...]
        parts = [jnp.max(jnp.where(bv == g, h, -jnp.inf), axis=0, keepdims=True)
                 for g in range(_G)]
        g_ref[...] = jnp.maximum(g_ref[...], jnp.concatenate(parts, axis=0))

        @pl.when(i == _NB - 1)
        def _finish():
            gg = g_ref[...]
            z = jnp.maximum(
                jnp.dot(gg, wl1[...], preferred_element_type=jnp.float32)
                + bl1[...], 0.0)
            o = jnp.dot(z, wl2[...], preferred_element_type=jnp.float32) + bl2[...]
            out_ref[...] = 1.0 / (1.0 + jnp.exp(-o))

    return pl.pallas_call(
        body,
        grid=(_NB,),
        in_specs=[
            pl.BlockSpec((_BLK, _D), lambda i: (i, 0)),
            pl.BlockSpec((_BLK, _D), lambda i: (i, 0)),
            pl.BlockSpec((_BLK, _D), lambda i: (i, 0)),
            pl.BlockSpec((_BLK, 1), lambda i: (i, 0)),
            pl.BlockSpec((1, _D), lambda i: (0, 0)),
            pl.BlockSpec((_BLK, 1), lambda i: (i, 0)),
            pl.BlockSpec((_D, 256), lambda i: (0, 0)),
            pl.BlockSpec((1, 256), lambda i: (0, 0)),
            pl.BlockSpec((256, 10), lambda i: (0, 0)),
            pl.BlockSpec((1, 10), lambda i: (0, 0)),
        ],
        out_specs=pl.BlockSpec((_G, 10), lambda i: (0, 0)),
        out_shape=jax.ShapeDtypeStruct((_G, 10), jnp.float32),
        scratch_shapes=[pltpu.VMEM((_G, _D), jnp.float32)],
    )


def kernel(x, edge_index, batch, W1, b1, W2, b2, W3, b3, Wl1, bl1, Wl2, bl2):
    f32 = jnp.float32
    x_p = jnp.pad(x, ((0, _NP - _N), (0, 0)))
    ei = edge_index.astype(jnp.int32)
    pad_e = jnp.full((_EP - _E,), _N, jnp.int32)
    src = jnp.concatenate([ei[0], pad_e]).reshape(_NW, _CH, _C)
    dst = jnp.concatenate([ei[1], pad_e]).reshape(_NW, _CH, _C)
    bat_p = jnp.concatenate(
        [batch.astype(jnp.int32), jnp.full((_NP - _N,), _G, jnp.int32)]
    ).reshape(_NP, 1)
    ones128 = jnp.ones((_C, _D), f32)
    z128 = jnp.zeros((_NP, _D), f32)

    W1p = jnp.pad(W1, ((0, 0), (0, _D - W1.shape[1])))
    W2p = jnp.pad(W2, ((0, _D - W2.shape[0]), (0, _D - W2.shape[1])))
    W3p = jnp.pad(W3, ((0, _D - W3.shape[0]), (0, _D - W3.shape[1])))
    Wl1p = jnp.pad(Wl1, ((0, _D - Wl1.shape[0]), (0, 0)))
    b1p = jnp.pad(b1, (0, _D - b1.shape[0])).reshape(1, _D)
    b2p = jnp.pad(b2, (0, _D - b2.shape[0])).reshape(1, _D)
    b3p = jnp.pad(b3, (0, _D - b3.shape[0])).reshape(1, _D)

    degp = _deg_kernel()(dst, ones128, z128)
    dinv, hs1 = _tc1()(degp[0], degp[1], x_p, W1p)
    p1 = _agg_kernel(16)(hs1, src, dst, z128)
    hs2 = _tc2()(p1[0], p1[1], hs1, dinv, b1p, W2p)
    p2 = _agg_kernel(32)(hs2, src, dst, z128)
    hs3 = _tc2()(p2[0], p2[1], hs2, dinv, b2p, W3p)
    p3 = _agg_kernel(64)(hs3, src, dst, z128)
    out = _tc3()(p3[0], p3[1], hs3, dinv, b3p, bat_p,
                 Wl1p, bl1.reshape(1, -1), Wl2, bl2.reshape(1, -1))
    return out


# speedup vs baseline: 17.4762x; 2.3686x over previous
import functools

import jax
import jax.numpy as jnp
from jax import lax
from jax.experimental import pallas as pl
from jax.experimental.pallas import tpu as pltpu
from jax.experimental.pallas import tpu_sc as plsc

_N = 10000
_E = 320000
_D = 128
_G = 16

_NP = 10240
_NC = 2
_NS = 16
_NW = _NC * _NS
_C = 128
_CH = 80
_EP = _NW * _CH * _C
_RPS = _NP // _NS
_R = 2
_HC = 40

_BLK = 1024
_NB = _NP // _BLK


def _mesh():
    return plsc.VectorSubcoreMesh(core_axis_name="c", subcore_axis_name="s",
                                  num_cores=_NC, num_subcores=_NS)


@functools.cache
def _deg_kernel():
    @functools.partial(
        pl.kernel,
        out_type=jax.ShapeDtypeStruct((_NC, _NP, 16), jnp.float32),
        mesh=_mesh(),
        scratch_types=[
            pltpu.VMEM((_CH, _C), jnp.int32),
            pltpu.VMEM((_C, 16), jnp.float32),
            pltpu.VMEM_SHARED((_NP, 16), jnp.float32),
            pltpu.SemaphoreType.DMA,
        ],
        compiler_params=pltpu.CompilerParams(use_tc_tiling_on_sc=False),
    )
    def deg_k(dst_hbm, ones_hbm, zeros_hbm, out_hbm, dst_v, ones_v, acc_sh, sem):
        cid = lax.axis_index("c")
        sid = lax.axis_index("s")
        wid = cid * _NS + sid
        r0 = sid * _RPS
        pltpu.sync_copy(zeros_hbm.at[pl.ds(r0, _RPS)], acc_sh.at[pl.ds(r0, _RPS)])
        pltpu.sync_copy(dst_hbm.at[wid], dst_v)
        pltpu.sync_copy(ones_hbm, ones_v)
        plsc.subcore_barrier()

        def body(ch, carry):
            pltpu.sync_copy(ones_v, acc_sh.at[dst_v.at[ch]], add=True)
            return carry

        lax.fori_loop(0, _CH, body, 0)
        plsc.subcore_barrier()
        pltpu.sync_copy(acc_sh.at[pl.ds(r0, _RPS)], out_hbm.at[cid, pl.ds(r0, _RPS)])

    return deg_k


@functools.cache
def _agg_kernel(d):
    @functools.partial(
        pl.kernel,
        out_type=jax.ShapeDtypeStruct((_NC, _NP, d), jnp.float32),
        mesh=_mesh(),
        scratch_types=[
            pltpu.VMEM((_HC, _C), jnp.int32),
            pltpu.VMEM((_HC, _C), jnp.int32),
            pltpu.VMEM((_R, _C, d), jnp.float32),
            pltpu.VMEM_SHARED((_NP, d), jnp.float32),
            [pltpu.SemaphoreType.DMA] * _R,
            [pltpu.SemaphoreType.DMA] * _R,
        ],
        compiler_params=pltpu.CompilerParams(use_tc_tiling_on_sc=False),
    )
    def agg_k(hs_hbm, src_hbm, dst_hbm, zeros_hbm, out_hbm,
              src_v, dst_v, rows_v, acc_sh, gsem, ssem):
        cid = lax.axis_index("c")
        sid = lax.axis_index("s")
        wid = cid * _NS + sid
        r0 = sid * _RPS
        pltpu.sync_copy(zeros_hbm.at[pl.ds(r0, _RPS)], acc_sh.at[pl.ds(r0, _RPS)])
        plsc.subcore_barrier()

        def gather(ch, b):
            return pltpu.async_copy(hs_hbm.at[src_v.at[ch]], rows_v.at[b], gsem[b])

        def scatter(ch, b):
            return pltpu.async_copy(rows_v.at[b], acc_sh.at[dst_v.at[ch]],
                                    ssem[b], add=True)

        for half in range(_CH // _HC):
            pltpu.sync_copy(src_hbm.at[wid, pl.ds(half * _HC, _HC)], src_v)
            pltpu.sync_copy(dst_hbm.at[wid, pl.ds(half * _HC, _HC)], dst_v)
            for b in range(_R):
                gather(b, b)

            def body(i, carry):
                for b in range(_R):
                    ch = i * _R + b
                    pltpu.make_async_copy(hs_hbm.at[src_v.at[ch]],
                                          rows_v.at[b], gsem[b]).wait()
                    scatter(ch, b)
                    pb = b - 1 if b else _R - 1
                    pch = ch - 1

                    @pl.when(pch >= 0)
                    def _():
                        pltpu.make_async_copy(
                            rows_v.at[pb],
                            acc_sh.at[dst_v.at[lax.max(pch, 0)]],
                            ssem[pb]).wait()

                        @pl.when(pch + _R < _HC)
                        def _():
                            gather(pch + _R, pb)
                return carry

            lax.fori_loop(0, _HC // _R, body, 0)
            pltpu.make_async_copy(rows_v.at[_R - 1],
                                  acc_sh.at[dst_v.at[_HC - 1]],
                                  ssem[_R - 1]).wait()
        plsc.subcore_barrier()
        pltpu.sync_copy(acc_sh.at[pl.ds(r0, _RPS)], out_hbm.at[cid, pl.ds(r0, _RPS)])

    return agg_k


@functools.cache
def _tc1():
    def body(p0, p1, x, w, dinv_ref, hs_ref):
        deg = p0[:, 0:1] + p1[:, 0:1] + 1.0
        dinv = 1.0 / jnp.sqrt(deg)
        dinv_ref[...] = dinv
        hs_ref[...] = dinv * jnp.dot(x[...], w[...],
                                     preferred_element_type=jnp.float32)

    return pl.pallas_call(
        body,
        grid=(_NB,),
        in_specs=[
            pl.BlockSpec((_BLK, 16), lambda i: (i, 0)),
            pl.BlockSpec((_BLK, 16), lambda i: (i, 0)),
            pl.BlockSpec((_BLK, _D), lambda i: (i, 0)),
            pl.BlockSpec((_D, 16), lambda i: (0, 0)),
        ],
        out_specs=[
            pl.BlockSpec((_BLK, 1), lambda i: (i, 0)),
            pl.BlockSpec((_BLK, 16), lambda i: (i, 0)),
        ],
        out_shape=[
            jax.ShapeDtypeStruct((_NP, 1), jnp.float32),
            jax.ShapeDtypeStruct((_NP, 16), jnp.float32),
        ],
    )


@functools.cache
def _tc2(di, do):
    def body(p0, p1, hs, dinv, b, w, out_ref):
        i = pl.program_id(0)
        rid = lax.broadcasted_iota(jnp.int32, (_BLK, 1), 0) + i * _BLK
        dv = dinv[...]
        h = dv * (p0[...] + p1[...] + hs[...]) + b[...]
        h = jnp.where(rid < _N, jnp.maximum(h, 0.0), 0.0)
        out_ref[...] = dv * jnp.dot(h, w[...], preferred_element_type=jnp.float32)

    return pl.pallas_call(
        body,
        grid=(_NB,),
        in_specs=[
            pl.BlockSpec((_BLK, di), lambda i: (i, 0)),
            pl.BlockSpec((_BLK, di), lambda i: (i, 0)),
            pl.BlockSpec((_BLK, di), lambda i: (i, 0)),
            pl.BlockSpec((_BLK, 1), lambda i: (i, 0)),
            pl.BlockSpec((1, di), lambda i: (0, 0)),
            pl.BlockSpec((di, do), lambda i: (0, 0)),
        ],
        out_specs=pl.BlockSpec((_BLK, do), lambda i: (i, 0)),
        out_shape=jax.ShapeDtypeStruct((_NP, do), jnp.float32),
    )


@functools.cache
def _tc3():
    def body(p0, p1, hs, dinv, b, bat, wl1, bl1, wl2, bl2, out_ref, g_ref):
        i = pl.program_id(0)

        @pl.when(i == 0)
        def _init():
            g_ref[...] = jnp.full((_G, 64), -jnp.inf, jnp.float32)

        rid = lax.broadcasted_iota(jnp.int32, (_BLK, 1), 0) + i * _BLK
        h = dinv[...] * (p0[...] + p1[...] + hs[...]) + b[...]
        h = jnp.where(rid < _N, jnp.maximum(h, 0.0), -jnp.inf)
        bv = bat[...]
        parts = [jnp.max(jnp.where(bv == g, h, -jnp.inf), axis=0, keepdims=True)
                 for g in range(_G)]
        g_ref[...] = jnp.maximum(g_ref[...], jnp.concatenate(parts, axis=0))

        @pl.when(i == _NB - 1)
        def _finish():
            gg = g_ref[...]
            z = jnp.maximum(
                jnp.dot(gg, wl1[...], preferred_element_type=jnp.float32)
                + bl1[...], 0.0)
            o = jnp.dot(z, wl2[...], preferred_element_type=jnp.float32) + bl2[...]
            out_ref[...] = 1.0 / (1.0 + jnp.exp(-o))

    return pl.pallas_call(
        body,
        grid=(_NB,),
        in_specs=[
            pl.BlockSpec((_BLK, 64), lambda i: (i, 0)),
            pl.BlockSpec((_BLK, 64), lambda i: (i, 0)),
            pl.BlockSpec((_BLK, 64), lambda i: (i, 0)),
            pl.BlockSpec((_BLK, 1), lambda i: (i, 0)),
            pl.BlockSpec((1, 64), lambda i: (0, 0)),
            pl.BlockSpec((_BLK, 1), lambda i: (i, 0)),
            pl.BlockSpec((64, 256), lambda i: (0, 0)),
            pl.BlockSpec((1, 256), lambda i: (0, 0)),
            pl.BlockSpec((256, 10), lambda i: (0, 0)),
            pl.BlockSpec((1, 10), lambda i: (0, 0)),
        ],
        out_specs=pl.BlockSpec((_G, 10), lambda i: (0, 0)),
        out_shape=jax.ShapeDtypeStruct((_G, 10), jnp.float32),
        scratch_shapes=[pltpu.VMEM((_G, 64), jnp.float32)],
    )


def kernel(x, edge_index, batch, W1, b1, W2, b2, W3, b3, Wl1, bl1, Wl2, bl2):
    f32 = jnp.float32
    x_p = jnp.pad(x, ((0, _NP - _N), (0, 0)))
    ei = edge_index.astype(jnp.int32)
    pad_e = jnp.full((_EP - _E,), _N, jnp.int32)
    src = jnp.concatenate([ei[0], pad_e]).reshape(_NW, _CH, _C)
    dst = jnp.concatenate([ei[1], pad_e]).reshape(_NW, _CH, _C)
    bat_p = jnp.concatenate(
        [batch.astype(jnp.int32), jnp.full((_NP - _N,), _G, jnp.int32)]
    ).reshape(_NP, 1)
    ones16 = jnp.ones((_C, 16), f32)
    z16 = jnp.zeros((_NP, 16), f32)
    z32 = jnp.zeros((_NP, 32), f32)
    z64 = jnp.zeros((_NP, 64), f32)

    degp = _deg_kernel()(dst, ones16, z16)
    dinv, hs1 = _tc1()(degp[0], degp[1], x_p, W1)
    p1 = _agg_kernel(16)(hs1, src, dst, z16)
    hs2 = _tc2(16, 32)(p1[0], p1[1], hs1, dinv, b1.reshape(1, -1), W2)
    p2 = _agg_kernel(32)(hs2, src, dst, z32)
    hs3 = _tc2(32, 64)(p2[0], p2[1], hs2, dinv, b2.reshape(1, -1), W3)
    p3 = _agg_kernel(64)(hs3, src, dst, z64)
    out = _tc3()(p3[0], p3[1], hs3, dinv, b3.reshape(1, -1), bat_p,
                 Wl1, bl1.reshape(1, -1), Wl2, bl2.reshape(1, -1))
    return out


# speedup vs baseline: 17.5558x; 1.0046x over previous
import functools

import jax
import jax.numpy as jnp
from jax import lax
from jax.experimental import pallas as pl
from jax.experimental.pallas import tpu as pltpu
from jax.experimental.pallas import tpu_sc as plsc

_N = 10000
_E = 320000
_D = 128
_G = 16

_NP = 10240
_NC = 2
_NS = 16
_NW = _NC * _NS
_C = 128
_CH = 80
_EP = _NW * _CH * _C
_RPS = _NP // _NS
_R = 2

_BLK = 1024
_NB = _NP // _BLK


def _mesh():
    return plsc.VectorSubcoreMesh(core_axis_name="c", subcore_axis_name="s",
                                  num_cores=_NC, num_subcores=_NS)


@functools.cache
def _deg_kernel():
    @functools.partial(
        pl.kernel,
        out_type=jax.ShapeDtypeStruct((_NC, _NP, 16), jnp.float32),
        mesh=_mesh(),
        scratch_types=[
            pltpu.VMEM((_CH, _C), jnp.int32),
            pltpu.VMEM((_C, 16), jnp.float32),
            pltpu.VMEM_SHARED((_NP, 16), jnp.float32),
            pltpu.SemaphoreType.DMA,
        ],
        compiler_params=pltpu.CompilerParams(use_tc_tiling_on_sc=False),
    )
    def deg_k(dst_hbm, ones_hbm, zeros_hbm, out_hbm, dst_v, ones_v, acc_sh, sem):
        cid = lax.axis_index("c")
        sid = lax.axis_index("s")
        wid = cid * _NS + sid
        r0 = sid * _RPS
        pltpu.sync_copy(zeros_hbm.at[pl.ds(r0, _RPS)], acc_sh.at[pl.ds(r0, _RPS)])
        pltpu.sync_copy(dst_hbm.at[wid], dst_v)
        pltpu.sync_copy(ones_hbm, ones_v)
        plsc.subcore_barrier()

        def body(ch, carry):
            pltpu.sync_copy(ones_v, acc_sh.at[dst_v.at[ch]], add=True)
            return carry

        lax.fori_loop(0, _CH, body, 0)
        plsc.subcore_barrier()
        pltpu.sync_copy(acc_sh.at[pl.ds(r0, _RPS)], out_hbm.at[cid, pl.ds(r0, _RPS)])

    return deg_k


@functools.cache
def _agg_kernel(d):
    @functools.partial(
        pl.kernel,
        out_type=jax.ShapeDtypeStruct((_NC, _NP, d), jnp.float32),
        mesh=_mesh(),
        scratch_types=[
            pltpu.VMEM((_CH, _C), jnp.int32),
            pltpu.VMEM((_CH, _C), jnp.int32),
            pltpu.VMEM((_R, _C, d), jnp.float32),
            pltpu.VMEM_SHARED((_NP, d), jnp.float32),
            [pltpu.SemaphoreType.DMA] * _R,
            [pltpu.SemaphoreType.DMA] * _R,
        ],
        compiler_params=pltpu.CompilerParams(use_tc_tiling_on_sc=False),
    )
    def agg_k(hs_hbm, src_hbm, dst_hbm, zeros_hbm, out_hbm,
              src_v, dst_v, rows_v, acc_sh, gsem, ssem):
        cid = lax.axis_index("c")
        sid = lax.axis_index("s")
        wid = cid * _NS + sid
        r0 = sid * _RPS
        pltpu.sync_copy(zeros_hbm.at[pl.ds(r0, _RPS)], acc_sh.at[pl.ds(r0, _RPS)])
        plsc.subcore_barrier()

        def gather(ch, b):
            return pltpu.async_copy(hs_hbm.at[src_v.at[ch]], rows_v.at[b], gsem[b])

        def scatter(ch, b):
            return pltpu.async_copy(rows_v.at[b], acc_sh.at[dst_v.at[ch]],
                                    ssem[b], add=True)

        pltpu.sync_copy(src_hbm.at[wid], src_v)
        pltpu.sync_copy(dst_hbm.at[wid], dst_v)
        for b in range(_R):
            gather(b, b)

        def body(i, carry):
            for b in range(_R):
                ch = i * _R + b
                pltpu.make_async_copy(hs_hbm.at[src_v.at[ch]],
                                      rows_v.at[b], gsem[b]).wait()
                scatter(ch, b)
                pb = b - 1 if b else _R - 1
                pch = ch - 1

                @pl.when(pch >= 0)
                def _():
                    pltpu.make_async_copy(
                        rows_v.at[pb],
                        acc_sh.at[dst_v.at[lax.max(pch, 0)]],
                        ssem[pb]).wait()

                    @pl.when(pch + _R < _CH)
                    def _():
                        gather(pch + _R, pb)
            return carry

        lax.fori_loop(0, _CH // _R, body, 0)
        pltpu.make_async_copy(rows_v.at[_R - 1],
                              acc_sh.at[dst_v.at[_CH - 1]],
                              ssem[_R - 1]).wait()
        plsc.subcore_barrier()
        pltpu.sync_copy(acc_sh.at[pl.ds(r0, _RPS)], out_hbm.at[cid, pl.ds(r0, _RPS)])

    return agg_k


@functools.cache
def _tc1():
    def body(p0, p1, x, w, dinv_ref, hs_ref):
        deg = p0[:, 0:1] + p1[:, 0:1] + 1.0
        dinv = 1.0 / jnp.sqrt(deg)
        dinv_ref[...] = dinv
        hs_ref[...] = dinv * jnp.dot(x[...], w[...],
                                     preferred_element_type=jnp.float32)

    return pl.pallas_call(
        body,
        grid=(_NB,),
        in_specs=[
            pl.BlockSpec((_BLK, 16), lambda i: (i, 0)),
            pl.BlockSpec((_BLK, 16), lambda i: (i, 0)),
            pl.BlockSpec((_BLK, _D), lambda i: (i, 0)),
            pl.BlockSpec((_D, 16), lambda i: (0, 0)),
        ],
        out_specs=[
            pl.BlockSpec((_BLK, 1), lambda i: (i, 0)),
            pl.BlockSpec((_BLK, 16), lambda i: (i, 0)),
        ],
        out_shape=[
            jax.ShapeDtypeStruct((_NP, 1), jnp.float32),
            jax.ShapeDtypeStruct((_NP, 16), jnp.float32),
        ],
    )


@functools.cache
def _tc2(di, do):
    def body(p0, p1, hs, dinv, b, w, out_ref):
        i = pl.program_id(0)
        rid = lax.broadcasted_iota(jnp.int32, (_BLK, 1), 0) + i * _BLK
        dv = dinv[...]
        h = dv * (p0[...] + p1[...] + hs[...]) + b[...]
        h = jnp.where(rid < _N, jnp.maximum(h, 0.0), 0.0)
        out_ref[...] = dv * jnp.dot(h, w[...], preferred_element_type=jnp.float32)

    return pl.pallas_call(
        body,
        grid=(_NB,),
        in_specs=[
            pl.BlockSpec((_BLK, di), lambda i: (i, 0)),
            pl.BlockSpec((_BLK, di), lambda i: (i, 0)),
            pl.BlockSpec((_BLK, di), lambda i: (i, 0)),
            pl.BlockSpec((_BLK, 1), lambda i: (i, 0)),
            pl.BlockSpec((1, di), lambda i: (0, 0)),
            pl.BlockSpec((di, do), lambda i: (0, 0)),
        ],
        out_specs=pl.BlockSpec((_BLK, do), lambda i: (i, 0)),
        out_shape=jax.ShapeDtypeStruct((_NP, do), jnp.float32),
    )


@functools.cache
def _tc3():
    def body(p0, p1, hs, dinv, b, bat, wl1, bl1, wl2, bl2, out_ref, g_ref):
        i = pl.program_id(0)

        @pl.when(i == 0)
        def _init():
            g_ref[...] = jnp.full((_G, 64), -jnp.inf, jnp.float32)

        rid = lax.broadcasted_iota(jnp.int32, (_BLK, 1), 0) + i * _BLK
        h = dinv[...] * (p0[...] + p1[...] + hs[...]) + b[...]
        h = jnp.where(rid < _N, jnp.maximum(h, 0.0), -jnp.inf)
        bv = bat[...]
        parts = [jnp.max(jnp.where(bv == g, h, -jnp.inf), axis=0, keepdims=True)
                 for g in range(_G)]
        g_ref[...] = jnp.maximum(g_ref[...], jnp.concatenate(parts, axis=0))

        @pl.when(i == _NB - 1)
        def _finish():
            gg = g_ref[...]
            z = jnp.maximum(
                jnp.dot(gg, wl1[...], preferred_element_type=jnp.float32)
                + bl1[...], 0.0)
            o = jnp.dot(z, wl2[...], preferred_element_type=jnp.float32) + bl2[...]
            out_ref[...] = 1.0 / (1.0 + jnp.exp(-o))

    return pl.pallas_call(
        body,
        grid=(_NB,),
        in_specs=[
            pl.BlockSpec((_BLK, 64), lambda i: (i, 0)),
            pl.BlockSpec((_BLK, 64), lambda i: (i, 0)),
            pl.BlockSpec((_BLK, 64), lambda i: (i, 0)),
            pl.BlockSpec((_BLK, 1), lambda i: (i, 0)),
            pl.BlockSpec((1, 64), lambda i: (0, 0)),
            pl.BlockSpec((_BLK, 1), lambda i: (i, 0)),
            pl.BlockSpec((64, 256), lambda i: (0, 0)),
            pl.BlockSpec((1, 256), lambda i: (0, 0)),
            pl.BlockSpec((256, 10), lambda i: (0, 0)),
            pl.BlockSpec((1, 10), lambda i: (0, 0)),
        ],
        out_specs=pl.BlockSpec((_G, 10), lambda i: (0, 0)),
        out_shape=jax.ShapeDtypeStruct((_G, 10), jnp.float32),
        scratch_shapes=[pltpu.VMEM((_G, 64), jnp.float32)],
    )


def kernel(x, edge_index, batch, W1, b1, W2, b2, W3, b3, Wl1, bl1, Wl2, bl2):
    f32 = jnp.float32
    x_p = jnp.pad(x, ((0, _NP - _N), (0, 0)))
    ei = edge_index.astype(jnp.int32)
    pad_e = jnp.full((_EP - _E,), _N, jnp.int32)
    src = jnp.concatenate([ei[0], pad_e]).reshape(_NW, _CH, _C)
    dst = jnp.concatenate([ei[1], pad_e]).reshape(_NW, _CH, _C)
    bat_p = jnp.concatenate(
        [batch.astype(jnp.int32), jnp.full((_NP - _N,), _G, jnp.int32)]
    ).reshape(_NP, 1)
    ones16 = jnp.ones((_C, 16), f32)
    z16 = jnp.zeros((_NP, 16), f32)
    z32 = jnp.zeros((_NP, 32), f32)
    z64 = jnp.zeros((_NP, 64), f32)

    degp = _deg_kernel()(dst, ones16, z16)
    dinv, hs1 = _tc1()(degp[0], degp[1], x_p, W1)
    p1 = _agg_kernel(16)(hs1, src, dst, z16)
    hs2 = _tc2(16, 32)(p1[0], p1[1], hs1, dinv, b1.reshape(1, -1), W2)
    p2 = _agg_kernel(32)(hs2, src, dst, z32)
    hs3 = _tc2(32, 64)(p2[0], p2[1], hs2, dinv, b2.reshape(1, -1), W3)
    p3 = _agg_kernel(64)(hs3, src, dst, z64)
    out = _tc3()(p3[0], p3[1], hs3, dinv, b3.reshape(1, -1), bat_p,
                 Wl1, bl1.reshape(1, -1), Wl2, bl2.reshape(1, -1))
    return out


# speedup vs baseline: 19.8324x; 1.1297x over previous
import functools

import jax
import jax.numpy as jnp
from jax import lax
from jax.experimental import pallas as pl
from jax.experimental.pallas import tpu as pltpu
from jax.experimental.pallas import tpu_sc as plsc

_N = 10000
_E = 320000
_D = 128
_G = 16

_NP = 10240
_NC = 2
_NS = 16
_NW = _NC * _NS
_C = 128
_CH = 80
_EP = _NW * _CH * _C
_RPS = _NP // _NS
_R = 4

_BLK = 1024
_NB = _NP // _BLK


def _mesh():
    return plsc.VectorSubcoreMesh(core_axis_name="c", subcore_axis_name="s",
                                  num_cores=_NC, num_subcores=_NS)


@functools.cache
def _deg_kernel():
    @functools.partial(
        pl.kernel,
        out_type=jax.ShapeDtypeStruct((_NC, _NP, 16), jnp.float32),
        mesh=_mesh(),
        scratch_types=[
            pltpu.VMEM((_CH, _C), jnp.int32),
            pltpu.VMEM((_C, 16), jnp.float32),
            pltpu.VMEM_SHARED((_NP, 16), jnp.float32),
            pltpu.SemaphoreType.DMA,
        ],
        compiler_params=pltpu.CompilerParams(use_tc_tiling_on_sc=False),
    )
    def deg_k(dst_hbm, ones_hbm, zeros_hbm, out_hbm, dst_v, ones_v, acc_sh, sem):
        cid = lax.axis_index("c")
        sid = lax.axis_index("s")
        wid = cid * _NS + sid
        r0 = sid * _RPS
        pltpu.sync_copy(zeros_hbm.at[pl.ds(r0, _RPS)], acc_sh.at[pl.ds(r0, _RPS)])
        pltpu.sync_copy(dst_hbm.at[wid], dst_v)
        pltpu.sync_copy(ones_hbm, ones_v)
        plsc.subcore_barrier()

        def body(ch, carry):
            pltpu.sync_copy(ones_v, acc_sh.at[dst_v.at[ch]], add=True)
            return carry

        lax.fori_loop(0, _CH, body, 0)
        plsc.subcore_barrier()
        pltpu.sync_copy(acc_sh.at[pl.ds(r0, _RPS)], out_hbm.at[cid, pl.ds(r0, _RPS)])

    return deg_k


@functools.cache
def _agg_kernel(d):
    @functools.partial(
        pl.kernel,
        out_type=jax.ShapeDtypeStruct((_NC, _NP, d), jnp.float32),
        mesh=_mesh(),
        scratch_types=[
            pltpu.VMEM((_CH, _C), jnp.int32),
            pltpu.VMEM((_CH, _C), jnp.int32),
            pltpu.VMEM((_R, _C, d), jnp.float32),
            pltpu.VMEM_SHARED((_NP, d), jnp.float32),
            [pltpu.SemaphoreType.DMA] * _R,
            [pltpu.SemaphoreType.DMA] * _R,
        ],
        compiler_params=pltpu.CompilerParams(use_tc_tiling_on_sc=False),
    )
    def agg_k(hs_hbm, src_hbm, dst_hbm, zeros_hbm, out_hbm,
              src_v, dst_v, rows_v, acc_sh, gsem, ssem):
        cid = lax.axis_index("c")
        sid = lax.axis_index("s")
        wid = cid * _NS + sid
        r0 = sid * _RPS
        pltpu.sync_copy(zeros_hbm.at[pl.ds(r0, _RPS)], acc_sh.at[pl.ds(r0, _RPS)])
        plsc.subcore_barrier()

        def gather(ch, b):
            return pltpu.async_copy(hs_hbm.at[src_v.at[ch]], rows_v.at[b], gsem[b])

        def scatter(ch, b):
            return pltpu.async_copy(rows_v.at[b], acc_sh.at[dst_v.at[ch]],
                                    ssem[b], add=True)

        pltpu.sync_copy(src_hbm.at[wid], src_v)
        pltpu.sync_copy(dst_hbm.at[wid], dst_v)
        for b in range(_R):
            gather(b, b)

        def body(i, carry):
            for b in range(_R):
                ch = i * _R + b
                pltpu.make_async_copy(hs_hbm.at[src_v.at[ch]],
                                      rows_v.at[b], gsem[b]).wait()
                scatter(ch, b)
                pb = b - 1 if b else _R - 1
                pch = ch - 1

                @pl.when(pch >= 0)
                def _():
                    pltpu.make_async_copy(
                        rows_v.at[pb],
                        acc_sh.at[dst_v.at[lax.max(pch, 0)]],
                        ssem[pb]).wait()

                    @pl.when(pch + _R < _CH)
                    def _():
                        gather(pch + _R, pb)
            return carry

        lax.fori_loop(0, _CH // _R, body, 0)
        pltpu.make_async_copy(rows_v.at[_R - 1],
                              acc_sh.at[dst_v.at[_CH - 1]],
                              ssem[_R - 1]).wait()
        plsc.subcore_barrier()
        pltpu.sync_copy(acc_sh.at[pl.ds(r0, _RPS)], out_hbm.at[cid, pl.ds(r0, _RPS)])

    return agg_k


@functools.cache
def _tc1():
    def body(p0, p1, x, w, dinv_ref, hs_ref):
        deg = p0[:, 0:1] + p1[:, 0:1] + 1.0
        dinv = 1.0 / jnp.sqrt(deg)
        dinv_ref[...] = dinv
        hs_ref[...] = dinv * jnp.dot(x[...], w[...],
                                     preferred_element_type=jnp.float32)

    return pl.pallas_call(
        body,
        grid=(_NB,),
        in_specs=[
            pl.BlockSpec((_BLK, 16), lambda i: (i, 0)),
            pl.BlockSpec((_BLK, 16), lambda i: (i, 0)),
            pl.BlockSpec((_BLK, _D), lambda i: (i, 0)),
            pl.BlockSpec((_D, 16), lambda i: (0, 0)),
        ],
        out_specs=[
            pl.BlockSpec((_BLK, 1), lambda i: (i, 0)),
            pl.BlockSpec((_BLK, 16), lambda i: (i, 0)),
        ],
        out_shape=[
            jax.ShapeDtypeStruct((_NP, 1), jnp.float32),
            jax.ShapeDtypeStruct((_NP, 16), jnp.float32),
        ],
    )


@functools.cache
def _tc2(di, do):
    def body(p0, p1, hs, dinv, b, w, out_ref):
        i = pl.program_id(0)
        rid = lax.broadcasted_iota(jnp.int32, (_BLK, 1), 0) + i * _BLK
        dv = dinv[...]
        h = dv * (p0[...] + p1[...] + hs[...]) + b[...]
        h = jnp.where(rid < _N, jnp.maximum(h, 0.0), 0.0)
        out_ref[...] = dv * jnp.dot(h, w[...], preferred_element_type=jnp.float32)

    return pl.pallas_call(
        body,
        grid=(_NB,),
        in_specs=[
            pl.BlockSpec((_BLK, di), lambda i: (i, 0)),
            pl.BlockSpec((_BLK, di), lambda i: (i, 0)),
            pl.BlockSpec((_BLK, di), lambda i: (i, 0)),
            pl.BlockSpec((_BLK, 1), lambda i: (i, 0)),
            pl.BlockSpec((1, di), lambda i: (0, 0)),
            pl.BlockSpec((di, do), lambda i: (0, 0)),
        ],
        out_specs=pl.BlockSpec((_BLK, do), lambda i: (i, 0)),
        out_shape=jax.ShapeDtypeStruct((_NP, do), jnp.float32),
    )


@functools.cache
def _tc3():
    def body(p0, p1, hs, dinv, b, bat, wl1, bl1, wl2, bl2, out_ref, g_ref):
        i = pl.program_id(0)

        @pl.when(i == 0)
        def _init():
            g_ref[...] = jnp.full((_G, 64), -jnp.inf, jnp.float32)

        rid = lax.broadcasted_iota(jnp.int32, (_BLK, 1), 0) + i * _BLK
        h = dinv[...] * (p0[...] + p1[...] + hs[...]) + b[...]
        h = jnp.where(rid < _N, jnp.maximum(h, 0.0), -jnp.inf)
        bv = bat[...]
        parts = [jnp.max(jnp.where(bv == g, h, -jnp.inf), axis=0, keepdims=True)
                 for g in range(_G)]
        g_ref[...] = jnp.maximum(g_ref[...], jnp.concatenate(parts, axis=0))

        @pl.when(i == _NB - 1)
        def _finish():
            gg = g_ref[...]
            z = jnp.maximum(
                jnp.dot(gg, wl1[...], preferred_element_type=jnp.float32)
                + bl1[...], 0.0)
            o = jnp.dot(z, wl2[...], preferred_element_type=jnp.float32) + bl2[...]
            out_ref[...] = 1.0 / (1.0 + jnp.exp(-o))

    return pl.pallas_call(
        body,
        grid=(_NB,),
        in_specs=[
            pl.BlockSpec((_BLK, 64), lambda i: (i, 0)),
            pl.BlockSpec((_BLK, 64), lambda i: (i, 0)),
            pl.BlockSpec((_BLK, 64), lambda i: (i, 0)),
            pl.BlockSpec((_BLK, 1), lambda i: (i, 0)),
            pl.BlockSpec((1, 64), lambda i: (0, 0)),
            pl.BlockSpec((_BLK, 1), lambda i: (i, 0)),
            pl.BlockSpec((64, 256), lambda i: (0, 0)),
            pl.BlockSpec((1, 256), lambda i: (0, 0)),
            pl.BlockSpec((256, 10), lambda i: (0, 0)),
            pl.BlockSpec((1, 10), lambda i: (0, 0)),
        ],
        out_specs=pl.BlockSpec((_G, 10), lambda i: (0, 0)),
        out_shape=jax.ShapeDtypeStruct((_G, 10), jnp.float32),
        scratch_shapes=[pltpu.VMEM((_G, 64), jnp.float32)],
    )


def kernel(x, edge_index, batch, W1, b1, W2, b2, W3, b3, Wl1, bl1, Wl2, bl2):
    f32 = jnp.float32
    x_p = jnp.pad(x, ((0, _NP - _N), (0, 0)))
    ei = edge_index.astype(jnp.int32)
    pad_e = jnp.full((_EP - _E,), _N, jnp.int32)
    src = jnp.concatenate([ei[0], pad_e]).reshape(_NW, _CH, _C)
    dst = jnp.concatenate([ei[1], pad_e]).reshape(_NW, _CH, _C)
    bat_p = jnp.concatenate(
        [batch.astype(jnp.int32), jnp.full((_NP - _N,), _G, jnp.int32)]
    ).reshape(_NP, 1)
    ones16 = jnp.ones((_C, 16), f32)
    z16 = jnp.zeros((_NP, 16), f32)
    z32 = jnp.zeros((_NP, 32), f32)
    z64 = jnp.zeros((_NP, 64), f32)

    degp = _deg_kernel()(dst, ones16, z16)
    dinv, hs1 = _tc1()(degp[0], degp[1], x_p, W1)
    p1 = _agg_kernel(16)(hs1, src, dst, z16)
    hs2 = _tc2(16, 32)(p1[0], p1[1], hs1, dinv, b1.reshape(1, -1), W2)
    p2 = _agg_kernel(32)(hs2, src, dst, z32)
    hs3 = _tc2(32, 64)(p2[0], p2[1], hs2, dinv, b2.reshape(1, -1), W3)
    p3 = _agg_kernel(64)(hs3, src, dst, z64)
    out = _tc3()(p3[0], p3[1], hs3, dinv, b3.reshape(1, -1), bat_p,
                 Wl1, bl1.reshape(1, -1), Wl2, bl2.reshape(1, -1))
    return out
